# baseline (speedup 1.0000x reference)
import functools

import jax
import jax.numpy as jnp
from jax import lax
from jax.experimental import pallas as pl
from jax.experimental.pallas import tpu as pltpu
from jax.experimental.pallas import tpu_sc as plsc

N = 884
FD = 128
E = 56576
K = 7
PR = 888
PW = PR * 128
NNF = K * PW
SLAB = K * PR
NS = 16
L = 16
EPT = E // NS
ZCH = NNF // NS
NIT = EPT // L


def _sc_body(eall, zeros_h, ones_h, out, src_v, dst_v, idx_v, ones_v, stage_v,
             acc, sem, fsem):
    c = lax.axis_index("c")
    s = lax.axis_index("s")

    z1 = pltpu.make_async_copy(zeros_h, stage_v, sem)
    z2 = pltpu.make_async_copy(ones_h, ones_v, sem)
    z1.start()
    z2.start()
    z1.wait()
    z2.wait()

    pltpu.sync_copy(stage_v, acc.at[pl.ds(s * ZCH, ZCH)])
    plsc.subcore_barrier()

    def histogram(eoff):
        base = eoff + s * EPT
        cp1 = pltpu.make_async_copy(eall.at[pl.ds(base, EPT)], src_v, sem)
        cp2 = pltpu.make_async_copy(eall.at[pl.ds(E + base, EPT)], dst_v, sem)
        cp1.start()
        cp2.start()
        cp1.wait()
        cp2.wait()

        def idx16(i16):
            sl = pl.ds(i16 * L, L)
            d = dst_v[sl]
            idx_v[sl] = ((d >> 7) * PW + (src_v[sl] << 7)) + (d & 127)

        def body(i, _):
            for j in range(4):
                idx16(i * 4 + j)
            return 0
        lax.fori_loop(0, NIT // 4, body, 0)
        for j in range((NIT // 4) * 4, NIT):
            idx16(j)
        pltpu.sync_copy(ones_v, acc.at[idx_v], add=True)

    histogram(c * (4 * E))
    plsc.subcore_barrier()

    pltpu.sync_copy(acc.at[pl.ds(s * ZCH, ZCH)], stage_v)
    plsc.subcore_barrier()

    vbase = c * (2 * NNF)
    flush = pltpu.make_async_copy(
        stage_v, out.at[pl.ds(vbase + s * ZCH, ZCH)], fsem)
    flush.start()

    @pl.when(c == 0)
    def _():
        histogram(2 * E)

    flush.wait()
    plsc.subcore_barrier()

    @pl.when(c == 0)
    def _():
        pltpu.sync_copy(acc.at[pl.ds(s * ZCH, ZCH)], stage_v)
        pltpu.sync_copy(stage_v, out.at[pl.ds(NNF + s * ZCH, ZCH)])


@functools.cache
def _sc_histogram():
    return pl.kernel(
        _sc_body,
        mesh=plsc.VectorSubcoreMesh(core_axis_name="c", subcore_axis_name="s"),
        out_type=jax.ShapeDtypeStruct((3 * NNF,), jnp.float32),
        scratch_types=[
            pltpu.VMEM((EPT,), jnp.int32),
            pltpu.VMEM((EPT,), jnp.int32),
            pltpu.VMEM((EPT,), jnp.int32),
            pltpu.VMEM((EPT,), jnp.float32),
            pltpu.VMEM((ZCH,), jnp.float32),
            pltpu.VMEM_SHARED((NNF,), jnp.float32),
            pltpu.SemaphoreType.DMA,
            pltpu.SemaphoreType.DMA,
        ],
    )


def _tc_body(cnt_hbm, dg, dc, dsm, x_ref,
             Wt1, Wt2, Ws1, Ws2, Wg1, Wg2,
             bt1, bt2, bs1, bs2, bg1, bg2,
             fc1W_ref, fc1b_ref, fc2W_ref, fc2b_ref, cnnW_ref, cnnb_ref,
             out_ref, cb0, cb1, cb2, db0, db1, db2, sem):
    cbufs = (cb0, cb1, cb2)
    dbufs = (db0, db1, db2)
    di_h = (dg, dc, dsm)
    descs = []
    for v in range(3):
        dc_ = pltpu.make_async_copy(
            cnt_hbm.at[pl.ds(v * SLAB, SLAB), :], cbufs[v], sem.at[v])
        dd_ = pltpu.make_async_copy(di_h[v], dbufs[v], sem.at[3 + v])
        dc_.start()
        dd_.start()
        descs.append((dc_, dd_))

    W1s = (Wt1, Ws1, Wg1)
    W2s = (Wt2, Ws2, Wg2)
    b1s = (bt1, bs1, bg1)
    b2s = (bt2, bs2, bg2)
    Xt = x_ref[...].T
    Zs = []
    P0 = None
    for v in range(3):
        descs[v][0].wait()
        descs[v][1].wait()
        Praw = [cbufs[v][pl.ds(k * PR, N), :] for k in range(K)]
        if v == 0:
            P0 = Praw
        P = [Praw[k] - P0[k] for k in range(K)] if v == 1 else Praw
        D = dbufs[v][...]
        Dp = jnp.concatenate(
            [D, jnp.zeros((N, K * 128 - N), jnp.float32)], axis=1)
        Bk = [P[k] * Dp[:, k * 128:(k + 1) * 128] for k in range(K)]
        deg = jnp.concatenate(
            [jnp.sum(Bk[k], axis=0, keepdims=True) for k in range(K)],
            axis=1)[:, :N] + 1.0
        dinv = lax.rsqrt(deg)
        G = jnp.dot(W1s[v][...].T, Xt,
                    preferred_element_type=jnp.float32) * dinv
        GB = jnp.concatenate(
            [jnp.dot(G, Bk[k], preferred_element_type=jnp.float32)
             for k in range(K)], axis=1)[:, :N]
        Z1 = jnp.maximum(dinv * (GB + G) + b1s[v][...], 0.0)
        G2 = jnp.dot(W2s[v][...].T, Z1,
                     preferred_element_type=jnp.float32) * dinv
        GB2 = jnp.concatenate(
            [jnp.dot(G2, Bk[k], preferred_element_type=jnp.float32)
             for k in range(K)], axis=1)[:, :N]
        Z2 = jnp.maximum(dinv * (GB2 + G2) + b2s[v][...], 0.0)
        Zs += [Z1, Z2]

    inv = 1.0 / (N * FD)
    fc1W = fc1W_ref[...]
    h1 = fc1b_ref[...]
    for cc in range(6):
        h1 = h1 + (jnp.sum(Zs[cc]) * inv) * fc1W[cc:cc + 1, :]
    h1 = jnp.maximum(h1, 0.0)
    h2 = jnp.dot(h1, fc2W_ref[...],
                 preferred_element_type=jnp.float32) + fc2b_ref[...]
    att = 1.0 / (1.0 + jnp.exp(-h2))
    coef = att * cnnW_ref[...]

    acc = coef[0, 0] * Zs[0]
    for cc in range(1, 6):
        acc = acc + coef[0, cc] * Zs[cc]
    out_ref[...] = acc.T + cnnb_ref[0, 0]


def kernel(x_d, di_gua, di_cos, di_sem, W_t1, b_t1, W_t2, b_t2, W_s1, b_s1,
           W_s2, b_s2, W_g1, b_g1, W_g2, b_g2, fc1_W, fc1_b, fc2_W, fc2_b,
           cnn_W, cnn_b, di_gua_edges, di_cos_edges, di_sem_edges):
    eall = jnp.concatenate([di_gua_edges.reshape(-1), di_cos_edges.reshape(-1),
                            di_sem_edges.reshape(-1)])
    counts = _sc_histogram()(eall, jnp.zeros((ZCH,), jnp.float32),
                             jnp.ones((EPT,), jnp.float32))
    counts = counts.reshape(3 * SLAB, 128)
    anyspec = pl.BlockSpec(memory_space=pl.ANY)
    vspec = pl.BlockSpec(memory_space=pltpu.MemorySpace.VMEM)
    out = pl.pallas_call(
        _tc_body,
        out_shape=jax.ShapeDtypeStruct((N, FD), jnp.float32),
        in_specs=[anyspec] * 4 + [vspec] * 19,
        out_specs=vspec,
        scratch_shapes=(
            [pltpu.VMEM((SLAB, 128), jnp.float32)] * 3
            + [pltpu.VMEM((N, N), jnp.float32)] * 3
            + [pltpu.SemaphoreType.DMA((6,))]
        ),
    )(counts, di_gua, di_cos, di_sem, x_d,
      W_t1, W_t2, W_s1, W_s2, W_g1, W_g2,
      b_t1.reshape(FD, 1), b_t2.reshape(FD, 1), b_s1.reshape(FD, 1),
      b_s2.reshape(FD, 1), b_g1.reshape(FD, 1), b_g2.reshape(FD, 1),
      fc1_W, fc1_b.reshape(1, -1), fc2_W, fc2_b.reshape(1, -1),
      cnn_W.reshape(1, -1), cnn_b.reshape(1, 1))
    return out


# speedup vs baseline: 1.1617x; 1.1617x over previous
import functools

import jax
import jax.numpy as jnp
from jax import lax
from jax.experimental import pallas as pl
from jax.experimental.pallas import tpu as pltpu
from jax.experimental.pallas import tpu_sc as plsc

N = 884
FD = 128
E = 56576
K = 7
PR = 888
PW = PR * 128
NNF = K * PW
SLAB = K * PR
NS = 16
L = 16
EPT = E // NS
ZCH = NNF // NS
NIT = EPT // L


def _sc_body(eall, out, src_v, dst_v, idx_v, ones_v, stage_v,
             acc, sem, fsem):
    c = lax.axis_index("c")
    s = lax.axis_index("s")
    zero16 = jnp.zeros((L,), jnp.float32)
    one16 = jnp.ones((L,), jnp.float32)
    nz = ZCH // L

    def fillz(i, _):
        for j in range(8):
            stage_v[pl.ds((i * 8 + j) * L, L)] = zero16
        return 0
    lax.fori_loop(0, nz // 8, fillz, 0)
    for j in range((nz // 8) * 8, nz):
        stage_v[pl.ds(j * L, L)] = zero16

    def fillo(i, _):
        for j in range(8):
            ones_v[pl.ds((i * 8 + j) * L, L)] = one16
        return 0
    lax.fori_loop(0, NIT // 8, fillo, 0)
    for j in range((NIT // 8) * 8, NIT):
        ones_v[pl.ds(j * L, L)] = one16

    pltpu.sync_copy(stage_v, acc.at[pl.ds(s * ZCH, ZCH)])
    plsc.subcore_barrier()

    def histogram(eoff):
        base = eoff + s * EPT
        cp1 = pltpu.make_async_copy(eall.at[pl.ds(base, EPT)], src_v, sem)
        cp2 = pltpu.make_async_copy(eall.at[pl.ds(E + base, EPT)], dst_v, sem)
        cp1.start()
        cp2.start()
        cp1.wait()
        cp2.wait()

        def idx16(i16):
            sl = pl.ds(i16 * L, L)
            d = dst_v[sl]
            idx_v[sl] = ((d >> 7) * PW + (src_v[sl] << 7)) + (d & 127)

        def body(i, _):
            for j in range(4):
                idx16(i * 4 + j)
            return 0
        lax.fori_loop(0, NIT // 4, body, 0)
        for j in range((NIT // 4) * 4, NIT):
            idx16(j)
        pltpu.sync_copy(ones_v, acc.at[idx_v], add=True)

    histogram(c * (4 * E))
    plsc.subcore_barrier()

    pltpu.sync_copy(acc.at[pl.ds(s * ZCH, ZCH)], stage_v)
    plsc.subcore_barrier()

    vbase = c * (2 * NNF)
    flush = pltpu.make_async_copy(
        stage_v, out.at[pl.ds(vbase + s * ZCH, ZCH)], fsem)
    flush.start()

    @pl.when(c == 0)
    def _():
        histogram(2 * E)

    flush.wait()
    plsc.subcore_barrier()

    @pl.when(c == 0)
    def _():
        pltpu.sync_copy(acc.at[pl.ds(s * ZCH, ZCH)], stage_v)
        pltpu.sync_copy(stage_v, out.at[pl.ds(NNF + s * ZCH, ZCH)])


@functools.cache
def _sc_histogram():
    return pl.kernel(
        _sc_body,
        mesh=plsc.VectorSubcoreMesh(core_axis_name="c", subcore_axis_name="s"),
        out_type=jax.ShapeDtypeStruct((3 * NNF,), jnp.float32),
        scratch_types=[
            pltpu.VMEM((EPT,), jnp.int32),
            pltpu.VMEM((EPT,), jnp.int32),
            pltpu.VMEM((EPT,), jnp.int32),
            pltpu.VMEM((EPT,), jnp.float32),
            pltpu.VMEM((ZCH,), jnp.float32),
            pltpu.VMEM_SHARED((NNF,), jnp.float32),
            pltpu.SemaphoreType.DMA,
            pltpu.SemaphoreType.DMA,
        ],
    )


def _tc_body(cnt_hbm, dg, dc, dsm, x_ref,
             Wt1, Wt2, Ws1, Ws2, Wg1, Wg2,
             bt1, bt2, bs1, bs2, bg1, bg2,
             fc1W_ref, fc1b_ref, fc2W_ref, fc2b_ref, cnnW_ref, cnnb_ref,
             out_ref, cb0, cb1, cb2, db0, db1, db2, sem):
    cbufs = (cb0, cb1, cb2)
    dbufs = (db0, db1, db2)
    di_h = (dg, dc, dsm)
    descs = []
    for v in range(3):
        dc_ = pltpu.make_async_copy(
            cnt_hbm.at[pl.ds(v * SLAB, SLAB), :], cbufs[v], sem.at[v])
        dd_ = pltpu.make_async_copy(di_h[v], dbufs[v], sem.at[3 + v])
        dc_.start()
        dd_.start()
        descs.append((dc_, dd_))

    W1s = (Wt1, Ws1, Wg1)
    W2s = (Wt2, Ws2, Wg2)
    b1s = (bt1, bs1, bg1)
    b2s = (bt2, bs2, bg2)
    Xt = x_ref[...].T
    Zs = []
    P0 = None
    for v in range(3):
        descs[v][0].wait()
        descs[v][1].wait()
        Praw = [cbufs[v][pl.ds(k * PR, N), :] for k in range(K)]
        if v == 0:
            P0 = Praw
        P = [Praw[k] - P0[k] for k in range(K)] if v == 1 else Praw
        D = dbufs[v][...]
        Dp = jnp.concatenate(
            [D, jnp.zeros((N, K * 128 - N), jnp.float32)], axis=1)
        Bk = [P[k] * Dp[:, k * 128:(k + 1) * 128] for k in range(K)]
        deg = jnp.concatenate(
            [jnp.sum(Bk[k], axis=0, keepdims=True) for k in range(K)],
            axis=1)[:, :N] + 1.0
        dinv = lax.rsqrt(deg)
        G = jnp.dot(W1s[v][...].T, Xt,
                    preferred_element_type=jnp.float32) * dinv
        GB = jnp.concatenate(
            [jnp.dot(G, Bk[k], preferred_element_type=jnp.float32)
             for k in range(K)], axis=1)[:, :N]
        Z1 = jnp.maximum(dinv * (GB + G) + b1s[v][...], 0.0)
        G2 = jnp.dot(W2s[v][...].T, Z1,
                     preferred_element_type=jnp.float32) * dinv
        GB2 = jnp.concatenate(
            [jnp.dot(G2, Bk[k], preferred_element_type=jnp.float32)
             for k in range(K)], axis=1)[:, :N]
        Z2 = jnp.maximum(dinv * (GB2 + G2) + b2s[v][...], 0.0)
        Zs += [Z1, Z2]

    inv = 1.0 / (N * FD)
    fc1W = fc1W_ref[...]
    h1 = fc1b_ref[...]
    for cc in range(6):
        h1 = h1 + (jnp.sum(Zs[cc]) * inv) * fc1W[cc:cc + 1, :]
    h1 = jnp.maximum(h1, 0.0)
    h2 = jnp.dot(h1, fc2W_ref[...],
                 preferred_element_type=jnp.float32) + fc2b_ref[...]
    att = 1.0 / (1.0 + jnp.exp(-h2))
    coef = att * cnnW_ref[...]

    acc = coef[0, 0] * Zs[0]
    for cc in range(1, 6):
        acc = acc + coef[0, cc] * Zs[cc]
    out_ref[...] = acc.T + cnnb_ref[0, 0]


def kernel(x_d, di_gua, di_cos, di_sem, W_t1, b_t1, W_t2, b_t2, W_s1, b_s1,
           W_s2, b_s2, W_g1, b_g1, W_g2, b_g2, fc1_W, fc1_b, fc2_W, fc2_b,
           cnn_W, cnn_b, di_gua_edges, di_cos_edges, di_sem_edges):
    eall = jnp.concatenate([di_gua_edges.reshape(-1), di_cos_edges.reshape(-1),
                            di_sem_edges.reshape(-1)])
    counts = _sc_histogram()(eall)
    counts = counts.reshape(3 * SLAB, 128)
    anyspec = pl.BlockSpec(memory_space=pl.ANY)
    vspec = pl.BlockSpec(memory_space=pltpu.MemorySpace.VMEM)
    out = pl.pallas_call(
        _tc_body,
        out_shape=jax.ShapeDtypeStruct((N, FD), jnp.float32),
        in_specs=[anyspec] * 4 + [vspec] * 19,
        out_specs=vspec,
        scratch_shapes=(
            [pltpu.VMEM((SLAB, 128), jnp.float32)] * 3
            + [pltpu.VMEM((N, N), jnp.float32)] * 3
            + [pltpu.SemaphoreType.DMA((6,))]
        ),
    )(counts, di_gua, di_cos, di_sem, x_d,
      W_t1, W_t2, W_s1, W_s2, W_g1, W_g2,
      b_t1.reshape(FD, 1), b_t2.reshape(FD, 1), b_s1.reshape(FD, 1),
      b_s2.reshape(FD, 1), b_g1.reshape(FD, 1), b_g2.reshape(FD, 1),
      fc1_W, fc1_b.reshape(1, -1), fc2_W, fc2_b.reshape(1, -1),
      cnn_W.reshape(1, -1), cnn_b.reshape(1, 1))
    return out


# speedup vs baseline: 1.1698x; 1.0070x over previous
import functools

import jax
import jax.numpy as jnp
from jax import lax
from jax.experimental import pallas as pl
from jax.experimental.pallas import tpu as pltpu
from jax.experimental.pallas import tpu_sc as plsc

N = 884
FD = 128
E = 56576
K = 7
PR = 888
PW = PR * 128
NNF = K * PW
SLAB = K * PR
NS = 16
L = 16
EPT = E // NS
ZCH = NNF // NS
NIT = EPT // L


def _sc_body(eall, out, src_v, dst_v, idx_v, ones_v, stage_v,
             acc, sem, fsem):
    c = lax.axis_index("c")
    s = lax.axis_index("s")
    zero16 = jnp.zeros((L,), jnp.float32)
    one16 = jnp.ones((L,), jnp.float32)
    nz = ZCH // L

    def fillz(i, _):
        for j in range(8):
            stage_v[pl.ds((i * 8 + j) * L, L)] = zero16
        return 0
    lax.fori_loop(0, nz // 8, fillz, 0)
    for j in range((nz // 8) * 8, nz):
        stage_v[pl.ds(j * L, L)] = zero16

    def fillo(i, _):
        for j in range(8):
            ones_v[pl.ds((i * 8 + j) * L, L)] = one16
        return 0
    lax.fori_loop(0, NIT // 8, fillo, 0)
    for j in range((NIT // 8) * 8, NIT):
        ones_v[pl.ds(j * L, L)] = one16

    pltpu.sync_copy(stage_v, acc.at[pl.ds(s * ZCH, ZCH)])
    plsc.subcore_barrier()

    def histogram(eoff):
        base = eoff + s * EPT
        cp1 = pltpu.make_async_copy(eall.at[pl.ds(base, EPT)], src_v, sem)
        cp2 = pltpu.make_async_copy(eall.at[pl.ds(E + base, EPT)], dst_v, sem)
        cp1.start()
        cp2.start()
        cp1.wait()
        cp2.wait()

        def idx16(i16):
            sl = pl.ds(i16 * L, L)
            d = dst_v[sl]
            idx_v[sl] = ((d >> 7) * PW + (src_v[sl] << 7)) + (d & 127)

        def body(i, _):
            for j in range(4):
                idx16(i * 4 + j)
            return 0
        lax.fori_loop(0, NIT // 4, body, 0)
        for j in range((NIT // 4) * 4, NIT):
            idx16(j)
        pltpu.sync_copy(ones_v, acc.at[idx_v], add=True)

    histogram(c * (4 * E))
    plsc.subcore_barrier()

    pltpu.sync_copy(acc.at[pl.ds(s * ZCH, ZCH)], stage_v)
    plsc.subcore_barrier()

    vbase = c * (2 * NNF)
    flush = pltpu.make_async_copy(
        stage_v, out.at[pl.ds(vbase + s * ZCH, ZCH)], fsem)
    flush.start()

    @pl.when(c == 0)
    def _():
        histogram(2 * E)

    flush.wait()
    plsc.subcore_barrier()

    @pl.when(c == 0)
    def _():
        h = ZCH // 2
        pltpu.sync_copy(acc.at[pl.ds(s * ZCH, h)], stage_v.at[pl.ds(0, h)])
        fa = pltpu.make_async_copy(
            stage_v.at[pl.ds(0, h)], out.at[pl.ds(NNF + s * ZCH, h)], fsem)
        fa.start()
        pltpu.sync_copy(acc.at[pl.ds(s * ZCH + h, h)], stage_v.at[pl.ds(h, h)])
        fb = pltpu.make_async_copy(
            stage_v.at[pl.ds(h, h)], out.at[pl.ds(NNF + s * ZCH + h, h)], fsem)
        fb.start()
        fa.wait()
        fb.wait()


@functools.cache
def _sc_histogram():
    return pl.kernel(
        _sc_body,
        mesh=plsc.VectorSubcoreMesh(core_axis_name="c", subcore_axis_name="s"),
        out_type=jax.ShapeDtypeStruct((3 * NNF,), jnp.float32),
        scratch_types=[
            pltpu.VMEM((EPT,), jnp.int32),
            pltpu.VMEM((EPT,), jnp.int32),
            pltpu.VMEM((EPT,), jnp.int32),
            pltpu.VMEM((EPT,), jnp.float32),
            pltpu.VMEM((ZCH,), jnp.float32),
            pltpu.VMEM_SHARED((NNF,), jnp.float32),
            pltpu.SemaphoreType.DMA,
            pltpu.SemaphoreType.DMA,
        ],
    )


def _tc_body(cnt_hbm, dg, dc, dsm, x_ref,
             Wt1, Wt2, Ws1, Ws2, Wg1, Wg2,
             bt1, bt2, bs1, bs2, bg1, bg2,
             fc1W_ref, fc1b_ref, fc2W_ref, fc2b_ref, cnnW_ref, cnnb_ref,
             out_ref, cb0, cb1, cb2, db0, db1, db2, sem):
    cbufs = (cb0, cb1, cb2)
    dbufs = (db0, db1, db2)
    di_h = (dg, dc, dsm)
    descs = []
    for v in range(3):
        dc_ = pltpu.make_async_copy(
            cnt_hbm.at[pl.ds(v * SLAB, SLAB), :], cbufs[v], sem.at[v])
        dd_ = pltpu.make_async_copy(di_h[v], dbufs[v], sem.at[3 + v])
        dc_.start()
        dd_.start()
        descs.append((dc_, dd_))

    W1s = (Wt1, Ws1, Wg1)
    W2s = (Wt2, Ws2, Wg2)
    b1s = (bt1, bs1, bg1)
    b2s = (bt2, bs2, bg2)
    Xt = x_ref[...].T
    Zs = []
    P0 = None
    for v in range(3):
        descs[v][0].wait()
        descs[v][1].wait()
        Praw = [cbufs[v][pl.ds(k * PR, N), :] for k in range(K)]
        if v == 0:
            P0 = Praw
        P = [Praw[k] - P0[k] for k in range(K)] if v == 1 else Praw
        D = dbufs[v][...]
        Dp = jnp.concatenate(
            [D, jnp.zeros((N, K * 128 - N), jnp.float32)], axis=1)
        Bk = [P[k] * Dp[:, k * 128:(k + 1) * 128] for k in range(K)]
        deg = jnp.concatenate(
            [jnp.sum(Bk[k], axis=0, keepdims=True) for k in range(K)],
            axis=1)[:, :N] + 1.0
        dinv = lax.rsqrt(deg)
        G = jnp.dot(W1s[v][...].T, Xt,
                    preferred_element_type=jnp.float32) * dinv
        GB = jnp.concatenate(
            [jnp.dot(G, Bk[k], preferred_element_type=jnp.float32)
             for k in range(K)], axis=1)[:, :N]
        Z1 = jnp.maximum(dinv * (GB + G) + b1s[v][...], 0.0)
        G2 = jnp.dot(W2s[v][...].T, Z1,
                     preferred_element_type=jnp.float32) * dinv
        GB2 = jnp.concatenate(
            [jnp.dot(G2, Bk[k], preferred_element_type=jnp.float32)
             for k in range(K)], axis=1)[:, :N]
        Z2 = jnp.maximum(dinv * (GB2 + G2) + b2s[v][...], 0.0)
        Zs += [Z1, Z2]

    inv = 1.0 / (N * FD)
    fc1W = fc1W_ref[...]
    h1 = fc1b_ref[...]
    for cc in range(6):
        h1 = h1 + (jnp.sum(Zs[cc]) * inv) * fc1W[cc:cc + 1, :]
    h1 = jnp.maximum(h1, 0.0)
    h2 = jnp.dot(h1, fc2W_ref[...],
                 preferred_element_type=jnp.float32) + fc2b_ref[...]
    att = 1.0 / (1.0 + jnp.exp(-h2))
    coef = att * cnnW_ref[...]

    acc = coef[0, 0] * Zs[0]
    for cc in range(1, 6):
        acc = acc + coef[0, cc] * Zs[cc]
    out_ref[...] = acc.T + cnnb_ref[0, 0]


def kernel(x_d, di_gua, di_cos, di_sem, W_t1, b_t1, W_t2, b_t2, W_s1, b_s1,
           W_s2, b_s2, W_g1, b_g1, W_g2, b_g2, fc1_W, fc1_b, fc2_W, fc2_b,
           cnn_W, cnn_b, di_gua_edges, di_cos_edges, di_sem_edges):
    eall = jnp.concatenate([di_gua_edges.reshape(-1), di_cos_edges.reshape(-1),
                            di_sem_edges.reshape(-1)])
    counts = _sc_histogram()(eall)
    counts = counts.reshape(3 * SLAB, 128)
    anyspec = pl.BlockSpec(memory_space=pl.ANY)
    vspec = pl.BlockSpec(memory_space=pltpu.MemorySpace.VMEM)
    out = pl.pallas_call(
        _tc_body,
        out_shape=jax.ShapeDtypeStruct((N, FD), jnp.float32),
        in_specs=[anyspec] * 4 + [vspec] * 19,
        out_specs=vspec,
        scratch_shapes=(
            [pltpu.VMEM((SLAB, 128), jnp.float32)] * 3
            + [pltpu.VMEM((N, N), jnp.float32)] * 3
            + [pltpu.SemaphoreType.DMA((6,))]
        ),
    )(counts, di_gua, di_cos, di_sem, x_d,
      W_t1, W_t2, W_s1, W_s2, W_g1, W_g2,
      b_t1.reshape(FD, 1), b_t2.reshape(FD, 1), b_s1.reshape(FD, 1),
      b_s2.reshape(FD, 1), b_g1.reshape(FD, 1), b_g2.reshape(FD, 1),
      fc1_W, fc1_b.reshape(1, -1), fc2_W, fc2_b.reshape(1, -1),
      cnn_W.reshape(1, -1), cnn_b.reshape(1, 1))
    return out


# speedup vs baseline: 1.2482x; 1.0671x over previous
import functools

import jax
import jax.numpy as jnp
from jax import lax
from jax.experimental import pallas as pl
from jax.experimental.pallas import tpu as pltpu
from jax.experimental.pallas import tpu_sc as plsc

N = 884
FD = 128
E = 56576
K = 7
PR = 888
PW = PR * 128
NNF = K * PW
SLAB = K * PR
NS = 16
L = 16
EPT = E // NS
ZCH = NNF // NS
NIT = EPT // L


def _sc_body(eall, out, src_v, dst_v, idx_v, ones_v, stage_v,
             acc, sem, fsem):
    c = lax.axis_index("c")
    s = lax.axis_index("s")
    zero16 = jnp.zeros((L,), jnp.float32)
    one16 = jnp.ones((L,), jnp.float32)
    nz = ZCH // L

    def fillz(i, _):
        for j in range(8):
            stage_v[pl.ds((i * 8 + j) * L, L)] = zero16
        return 0
    lax.fori_loop(0, nz // 8, fillz, 0)
    for j in range((nz // 8) * 8, nz):
        stage_v[pl.ds(j * L, L)] = zero16

    def fillo(i, _):
        for j in range(8):
            ones_v[pl.ds((i * 8 + j) * L, L)] = one16
        return 0
    lax.fori_loop(0, NIT // 8, fillo, 0)
    for j in range((NIT // 8) * 8, NIT):
        ones_v[pl.ds(j * L, L)] = one16

    def edges_descs(eoff):
        base = eoff + s * EPT
        return (pltpu.make_async_copy(eall.at[pl.ds(base, EPT)], src_v, sem),
                pltpu.make_async_copy(eall.at[pl.ds(E + base, EPT)], dst_v,
                                      sem))

    def start(cps):
        cps[0].start()
        cps[1].start()

    def compute_idx(cps):
        cps[0].wait()
        cps[1].wait()

        def idx16(i16):
            sl = pl.ds(i16 * L, L)
            d = dst_v[sl]
            idx_v[sl] = ((d >> 7) * PW + (src_v[sl] << 7)) + (d & 127)

        def body(i, _):
            for j in range(4):
                idx16(i * 4 + j)
            return 0
        lax.fori_loop(0, NIT // 4, body, 0)
        for j in range((NIT // 4) * 4, NIT):
            idx16(j)

    def scatter():
        pltpu.sync_copy(ones_v, acc.at[idx_v], add=True)

    cps1 = edges_descs(c * (4 * E))
    cps2 = edges_descs(2 * E)
    start(cps1)

    pltpu.sync_copy(stage_v, acc.at[pl.ds(s * ZCH, ZCH)])
    plsc.subcore_barrier()

    compute_idx(cps1)

    @pl.when(c == 0)
    def _():
        start(cps2)

    scatter()
    plsc.subcore_barrier()

    pltpu.sync_copy(acc.at[pl.ds(s * ZCH, ZCH)], stage_v)
    plsc.subcore_barrier()

    vbase = c * (2 * NNF)
    flush = pltpu.make_async_copy(
        stage_v, out.at[pl.ds(vbase + s * ZCH, ZCH)], fsem)
    flush.start()

    @pl.when(c == 0)
    def _():
        compute_idx(cps2)
        scatter()

    flush.wait()
    plsc.subcore_barrier()

    @pl.when(c == 0)
    def _():
        h = ZCH // 2
        pltpu.sync_copy(acc.at[pl.ds(s * ZCH, h)], stage_v.at[pl.ds(0, h)])
        fa = pltpu.make_async_copy(
            stage_v.at[pl.ds(0, h)], out.at[pl.ds(NNF + s * ZCH, h)], fsem)
        fa.start()
        pltpu.sync_copy(acc.at[pl.ds(s * ZCH + h, h)], stage_v.at[pl.ds(h, h)])
        fb = pltpu.make_async_copy(
            stage_v.at[pl.ds(h, h)], out.at[pl.ds(NNF + s * ZCH + h, h)], fsem)
        fb.start()
        fa.wait()
        fb.wait()


@functools.cache
def _sc_histogram():
    return pl.kernel(
        _sc_body,
        mesh=plsc.VectorSubcoreMesh(core_axis_name="c", subcore_axis_name="s"),
        out_type=jax.ShapeDtypeStruct((3 * NNF,), jnp.float32),
        scratch_types=[
            pltpu.VMEM((EPT,), jnp.int32),
            pltpu.VMEM((EPT,), jnp.int32),
            pltpu.VMEM((EPT,), jnp.int32),
            pltpu.VMEM((EPT,), jnp.float32),
            pltpu.VMEM((ZCH,), jnp.float32),
            pltpu.VMEM_SHARED((NNF,), jnp.float32),
            pltpu.SemaphoreType.DMA,
            pltpu.SemaphoreType.DMA,
        ],
    )


def _tc_body(cnt_hbm, dg, dc, dsm, x_ref,
             Wt1, Wt2, Ws1, Ws2, Wg1, Wg2,
             bt1, bt2, bs1, bs2, bg1, bg2,
             fc1W_ref, fc1b_ref, fc2W_ref, fc2b_ref, cnnW_ref, cnnb_ref,
             out_ref, cb0, cb1, cb2, db0, db1, db2, sem):
    cbufs = (cb0, cb1, cb2)
    dbufs = (db0, db1, db2)
    di_h = (dg, dc, dsm)
    descs = []
    for v in range(3):
        dc_ = pltpu.make_async_copy(
            cnt_hbm.at[pl.ds(v * SLAB, SLAB), :], cbufs[v], sem.at[v])
        dd_ = pltpu.make_async_copy(di_h[v], dbufs[v], sem.at[3 + v])
        dc_.start()
        dd_.start()
        descs.append((dc_, dd_))

    W1s = (Wt1, Ws1, Wg1)
    W2s = (Wt2, Ws2, Wg2)
    b1s = (bt1, bs1, bg1)
    b2s = (bt2, bs2, bg2)
    Xt = x_ref[...].T
    Zs = []
    P0 = None
    for v in range(3):
        descs[v][0].wait()
        descs[v][1].wait()
        Praw = [cbufs[v][pl.ds(k * PR, N), :] for k in range(K)]
        if v == 0:
            P0 = Praw
        P = [Praw[k] - P0[k] for k in range(K)] if v == 1 else Praw
        D = dbufs[v][...]
        Dp = jnp.concatenate(
            [D, jnp.zeros((N, K * 128 - N), jnp.float32)], axis=1)
        Bk = [P[k] * Dp[:, k * 128:(k + 1) * 128] for k in range(K)]
        deg = jnp.concatenate(
            [jnp.sum(Bk[k], axis=0, keepdims=True) for k in range(K)],
            axis=1)[:, :N] + 1.0
        dinv = lax.rsqrt(deg)
        G = jnp.dot(W1s[v][...].T, Xt,
                    preferred_element_type=jnp.float32) * dinv
        GB = jnp.concatenate(
            [jnp.dot(G, Bk[k], preferred_element_type=jnp.float32)
             for k in range(K)], axis=1)[:, :N]
        Z1 = jnp.maximum(dinv * (GB + G) + b1s[v][...], 0.0)
        G2 = jnp.dot(W2s[v][...].T, Z1,
                     preferred_element_type=jnp.float32) * dinv
        GB2 = jnp.concatenate(
            [jnp.dot(G2, Bk[k], preferred_element_type=jnp.float32)
             for k in range(K)], axis=1)[:, :N]
        Z2 = jnp.maximum(dinv * (GB2 + G2) + b2s[v][...], 0.0)
        Zs += [Z1, Z2]

    inv = 1.0 / (N * FD)
    fc1W = fc1W_ref[...]
    h1 = fc1b_ref[...]
    for cc in range(6):
        h1 = h1 + (jnp.sum(Zs[cc]) * inv) * fc1W[cc:cc + 1, :]
    h1 = jnp.maximum(h1, 0.0)
    h2 = jnp.dot(h1, fc2W_ref[...],
                 preferred_element_type=jnp.float32) + fc2b_ref[...]
    att = 1.0 / (1.0 + jnp.exp(-h2))
    coef = att * cnnW_ref[...]

    acc = coef[0, 0] * Zs[0]
    for cc in range(1, 6):
        acc = acc + coef[0, cc] * Zs[cc]
    out_ref[...] = acc.T + cnnb_ref[0, 0]


def kernel(x_d, di_gua, di_cos, di_sem, W_t1, b_t1, W_t2, b_t2, W_s1, b_s1,
           W_s2, b_s2, W_g1, b_g1, W_g2, b_g2, fc1_W, fc1_b, fc2_W, fc2_b,
           cnn_W, cnn_b, di_gua_edges, di_cos_edges, di_sem_edges):
    eall = jnp.concatenate([di_gua_edges.reshape(-1), di_cos_edges.reshape(-1),
                            di_sem_edges.reshape(-1)])
    counts = _sc_histogram()(eall)
    counts = counts.reshape(3 * SLAB, 128)
    anyspec = pl.BlockSpec(memory_space=pl.ANY)
    vspec = pl.BlockSpec(memory_space=pltpu.MemorySpace.VMEM)
    out = pl.pallas_call(
        _tc_body,
        out_shape=jax.ShapeDtypeStruct((N, FD), jnp.float32),
        in_specs=[anyspec] * 4 + [vspec] * 19,
        out_specs=vspec,
        scratch_shapes=(
            [pltpu.VMEM((SLAB, 128), jnp.float32)] * 3
            + [pltpu.VMEM((N, N), jnp.float32)] * 3
            + [pltpu.SemaphoreType.DMA((6,))]
        ),
    )(counts, di_gua, di_cos, di_sem, x_d,
      W_t1, W_t2, W_s1, W_s2, W_g1, W_g2,
      b_t1.reshape(FD, 1), b_t2.reshape(FD, 1), b_s1.reshape(FD, 1),
      b_s2.reshape(FD, 1), b_g1.reshape(FD, 1), b_g2.reshape(FD, 1),
      fc1_W, fc1_b.reshape(1, -1), fc2_W, fc2_b.reshape(1, -1),
      cnn_W.reshape(1, -1), cnn_b.reshape(1, 1))
    return out


# speedup vs baseline: 1.2851x; 1.0295x over previous
import functools

import jax
import jax.numpy as jnp
from jax import lax
from jax.experimental import pallas as pl
from jax.experimental.pallas import tpu as pltpu
from jax.experimental.pallas import tpu_sc as plsc

N = 884
FD = 128
E = 56576
K = 7
PR = 888
PW = PR * 128
NNF = K * PW
SLAB = K * PR
NS = 16
L = 16
EPT = E // NS
ZCH = NNF // NS
NIT = EPT // L


def _sc_body(eall, out, src_v, dst_v, idx_v, ones_v, stage_v,
             acc, sem, fsem):
    c = lax.axis_index("c")
    s = lax.axis_index("s")
    zero16 = jnp.zeros((L,), jnp.float32)
    one16 = jnp.ones((L,), jnp.float32)
    q = ZCH // 4
    nz = q // L

    def fillz(i, _):
        for j in range(8):
            stage_v[pl.ds((i * 8 + j) * L, L)] = zero16
        return 0
    lax.fori_loop(0, nz // 8, fillz, 0)
    for j in range((nz // 8) * 8, nz):
        stage_v[pl.ds(j * L, L)] = zero16

    zcs = [pltpu.make_async_copy(stage_v.at[pl.ds(0, q)],
                                 acc.at[pl.ds(s * ZCH + t * q, q)], fsem)
           for t in range(4)]
    for zc in zcs:
        zc.start()

    def fillo(i, _):
        for j in range(8):
            ones_v[pl.ds((i * 8 + j) * L, L)] = one16
        return 0
    lax.fori_loop(0, NIT // 8, fillo, 0)
    for j in range((NIT // 8) * 8, NIT):
        ones_v[pl.ds(j * L, L)] = one16

    def edges_descs(eoff):
        base = eoff + s * EPT
        return (pltpu.make_async_copy(eall.at[pl.ds(base, EPT)], src_v, sem),
                pltpu.make_async_copy(eall.at[pl.ds(E + base, EPT)], dst_v,
                                      sem))

    def start(cps):
        cps[0].start()
        cps[1].start()

    def compute_idx(cps):
        cps[0].wait()
        cps[1].wait()

        def idx16(i16):
            sl = pl.ds(i16 * L, L)
            d = dst_v[sl]
            idx_v[sl] = ((d >> 7) * PW + (src_v[sl] << 7)) + (d & 127)

        def body(i, _):
            for j in range(4):
                idx16(i * 4 + j)
            return 0
        lax.fori_loop(0, NIT // 4, body, 0)
        for j in range((NIT // 4) * 4, NIT):
            idx16(j)

    def scatter():
        pltpu.sync_copy(ones_v, acc.at[idx_v], add=True)

    cps1 = edges_descs(c * (4 * E))
    cps2 = edges_descs(2 * E)
    start(cps1)

    for zc in zcs:
        zc.wait()
    plsc.subcore_barrier()

    compute_idx(cps1)

    @pl.when(c == 0)
    def _():
        start(cps2)

    scatter()
    plsc.subcore_barrier()

    pltpu.sync_copy(acc.at[pl.ds(s * ZCH, ZCH)], stage_v)
    plsc.subcore_barrier()

    vbase = c * (2 * NNF)
    flush = pltpu.make_async_copy(
        stage_v, out.at[pl.ds(vbase + s * ZCH, ZCH)], fsem)
    flush.start()

    @pl.when(c == 0)
    def _():
        compute_idx(cps2)
        scatter()

    flush.wait()
    plsc.subcore_barrier()

    @pl.when(c == 0)
    def _():
        h = ZCH // 2
        pltpu.sync_copy(acc.at[pl.ds(s * ZCH, h)], stage_v.at[pl.ds(0, h)])
        fa = pltpu.make_async_copy(
            stage_v.at[pl.ds(0, h)], out.at[pl.ds(NNF + s * ZCH, h)], fsem)
        fa.start()
        pltpu.sync_copy(acc.at[pl.ds(s * ZCH + h, h)], stage_v.at[pl.ds(h, h)])
        fb = pltpu.make_async_copy(
            stage_v.at[pl.ds(h, h)], out.at[pl.ds(NNF + s * ZCH + h, h)], fsem)
        fb.start()
        fa.wait()
        fb.wait()


@functools.cache
def _sc_histogram():
    return pl.kernel(
        _sc_body,
        mesh=plsc.VectorSubcoreMesh(core_axis_name="c", subcore_axis_name="s"),
        out_type=jax.ShapeDtypeStruct((3 * NNF,), jnp.float32),
        scratch_types=[
            pltpu.VMEM((EPT,), jnp.int32),
            pltpu.VMEM((EPT,), jnp.int32),
            pltpu.VMEM((EPT,), jnp.int32),
            pltpu.VMEM((EPT,), jnp.float32),
            pltpu.VMEM((ZCH,), jnp.float32),
            pltpu.VMEM_SHARED((NNF,), jnp.float32),
            pltpu.SemaphoreType.DMA,
            pltpu.SemaphoreType.DMA,
        ],
    )


def _tc_body(cnt_hbm, dg, dc, dsm, x_ref,
             Wt1, Wt2, Ws1, Ws2, Wg1, Wg2,
             bt1, bt2, bs1, bs2, bg1, bg2,
             fc1W_ref, fc1b_ref, fc2W_ref, fc2b_ref, cnnW_ref, cnnb_ref,
             out_ref, cb0, cb1, cb2, db0, db1, db2, sem):
    cbufs = (cb0, cb1, cb2)
    dbufs = (db0, db1, db2)
    di_h = (dg, dc, dsm)
    descs = []
    for v in range(3):
        dc_ = pltpu.make_async_copy(
            cnt_hbm.at[pl.ds(v * SLAB, SLAB), :], cbufs[v], sem.at[v])
        dd_ = pltpu.make_async_copy(di_h[v], dbufs[v], sem.at[3 + v])
        dc_.start()
        dd_.start()
        descs.append((dc_, dd_))

    W1s = (Wt1, Ws1, Wg1)
    W2s = (Wt2, Ws2, Wg2)
    b1s = (bt1, bs1, bg1)
    b2s = (bt2, bs2, bg2)
    Xt = x_ref[...].T
    Zs = []
    P0 = None
    for v in range(3):
        descs[v][0].wait()
        descs[v][1].wait()
        Praw = [cbufs[v][pl.ds(k * PR, N), :] for k in range(K)]
        if v == 0:
            P0 = Praw
        P = [Praw[k] - P0[k] for k in range(K)] if v == 1 else Praw
        D = dbufs[v][...]
        Dp = jnp.concatenate(
            [D, jnp.zeros((N, K * 128 - N), jnp.float32)], axis=1)
        Bk = [P[k] * Dp[:, k * 128:(k + 1) * 128] for k in range(K)]
        deg = jnp.concatenate(
            [jnp.sum(Bk[k], axis=0, keepdims=True) for k in range(K)],
            axis=1)[:, :N] + 1.0
        dinv = lax.rsqrt(deg)
        G = jnp.dot(W1s[v][...].T, Xt,
                    preferred_element_type=jnp.float32) * dinv
        GB = jnp.concatenate(
            [jnp.dot(G, Bk[k], preferred_element_type=jnp.float32)
             for k in range(K)], axis=1)[:, :N]
        Z1 = jnp.maximum(dinv * (GB + G) + b1s[v][...], 0.0)
        G2 = jnp.dot(W2s[v][...].T, Z1,
                     preferred_element_type=jnp.float32) * dinv
        GB2 = jnp.concatenate(
            [jnp.dot(G2, Bk[k], preferred_element_type=jnp.float32)
             for k in range(K)], axis=1)[:, :N]
        Z2 = jnp.maximum(dinv * (GB2 + G2) + b2s[v][...], 0.0)
        Zs += [Z1, Z2]

    inv = 1.0 / (N * FD)
    fc1W = fc1W_ref[...]
    h1 = fc1b_ref[...]
    for cc in range(6):
        h1 = h1 + (jnp.sum(Zs[cc]) * inv) * fc1W[cc:cc + 1, :]
    h1 = jnp.maximum(h1, 0.0)
    h2 = jnp.dot(h1, fc2W_ref[...],
                 preferred_element_type=jnp.float32) + fc2b_ref[...]
    att = 1.0 / (1.0 + jnp.exp(-h2))
    coef = att * cnnW_ref[...]

    acc = coef[0, 0] * Zs[0]
    for cc in range(1, 6):
        acc = acc + coef[0, cc] * Zs[cc]
    out_ref[...] = acc.T + cnnb_ref[0, 0]


def kernel(x_d, di_gua, di_cos, di_sem, W_t1, b_t1, W_t2, b_t2, W_s1, b_s1,
           W_s2, b_s2, W_g1, b_g1, W_g2, b_g2, fc1_W, fc1_b, fc2_W, fc2_b,
           cnn_W, cnn_b, di_gua_edges, di_cos_edges, di_sem_edges):
    eall = jnp.concatenate([di_gua_edges.reshape(-1), di_cos_edges.reshape(-1),
                            di_sem_edges.reshape(-1)])
    counts = _sc_histogram()(eall)
    counts = counts.reshape(3 * SLAB, 128)
    anyspec = pl.BlockSpec(memory_space=pl.ANY)
    vspec = pl.BlockSpec(memory_space=pltpu.MemorySpace.VMEM)
    out = pl.pallas_call(
        _tc_body,
        out_shape=jax.ShapeDtypeStruct((N, FD), jnp.float32),
        in_specs=[anyspec] * 4 + [vspec] * 19,
        out_specs=vspec,
        scratch_shapes=(
            [pltpu.VMEM((SLAB, 128), jnp.float32)] * 3
            + [pltpu.VMEM((N, N), jnp.float32)] * 3
            + [pltpu.SemaphoreType.DMA((6,))]
        ),
    )(counts, di_gua, di_cos, di_sem, x_d,
      W_t1, W_t2, W_s1, W_s2, W_g1, W_g2,
      b_t1.reshape(FD, 1), b_t2.reshape(FD, 1), b_s1.reshape(FD, 1),
      b_s2.reshape(FD, 1), b_g1.reshape(FD, 1), b_g2.reshape(FD, 1),
      fc1_W, fc1_b.reshape(1, -1), fc2_W, fc2_b.reshape(1, -1),
      cnn_W.reshape(1, -1), cnn_b.reshape(1, 1))
    return out
